# Initial kernel scaffold; baseline (speedup 1.0000x reference)
import jax
import jax.numpy as jnp
from jax.experimental import pallas as pl


def kernel(x, edge_index, W1, a_src1, a_dst1, b1, W2, a_src2, a_dst2, b2):
    raise NotImplementedError("write your pallas kernel here")


# speedup vs baseline: 1.5078x; 1.5078x over previous
import jax
import jax.numpy as jnp
from jax.experimental import pallas as pl
from jax.experimental.pallas import tpu as pltpu

GRAPH_SIZE = 100
HID = 128


def _gat_layer(x, src, dst, W, a_s, a_d, b):
    n = x.shape[0]
    h = x @ W
    s = h @ a_s
    d = h @ a_d
    e = jax.nn.leaky_relu(s[src] + d[dst], negative_slope=0.2)
    ex = jnp.exp(e)
    denom = jax.ops.segment_sum(ex, dst, num_segments=n)
    num = jax.ops.segment_sum(h[src] * ex[:, None], dst, num_segments=n)
    return num / (denom[:, None] + 1e-16) + b


def _decoder_body(z_ref, out_ref):
    zb = z_ref[0]
    logits = jax.lax.dot_general(zb, zb, (((1,), (1,)), ((), ())),
                                 preferred_element_type=jnp.float32)
    m = jnp.max(logits, axis=-1, keepdims=True)
    e = jnp.exp(logits - m)
    out_ref[0] = e / jnp.sum(e, axis=-1, keepdims=True)


def kernel(x, edge_index, W1, a_src1, a_dst1, b1, W2, a_src2, a_dst2, b2):
    n = x.shape[0]
    src = edge_index[0].astype(jnp.int32)
    dst = edge_index[1].astype(jnp.int32)
    loop = jnp.arange(n, dtype=jnp.int32)
    src = jnp.concatenate([src, loop])
    dst = jnp.concatenate([dst, loop])
    z = jax.nn.relu(_gat_layer(x, src, dst, W1, a_src1, a_dst1, b1))
    z = jax.nn.relu(_gat_layer(z, src, dst, W2, a_src2, a_dst2, b2))
    B = n // GRAPH_SIZE
    zb = z.reshape(B, GRAPH_SIZE, HID)
    pi = pl.pallas_call(
        _decoder_body,
        grid=(B,),
        in_specs=[pl.BlockSpec((1, GRAPH_SIZE, HID), lambda b: (b, 0, 0))],
        out_specs=pl.BlockSpec((1, GRAPH_SIZE, GRAPH_SIZE), lambda b: (b, 0, 0)),
        out_shape=jax.ShapeDtypeStruct((B, GRAPH_SIZE, GRAPH_SIZE), jnp.float32),
    )(zb)
    return pi


# speedup vs baseline: 14.0956x; 9.3485x over previous
import functools

import jax
import jax.numpy as jnp
from jax import lax
from jax.experimental import pallas as pl
from jax.experimental.pallas import tpu as pltpu
from jax.experimental.pallas import tpu_sc as plsc

N = 10000
E = 320000
HID = 128
GRAPH_SIZE = 100
NB = N // GRAPH_SIZE

NTILES = 32
EPT = E // NTILES
CHUNK = 80
NCHUNK = EPT // CHUNK
NP = 10240
HALF = NP // 2
ACC = HALF + 128
ROWS_PT = HALF // 16
DROWS = 128


def _dense_body(x_ref, w_ref, as_ref, ad_ref, h_ref, s_ref, d_ref, sw_ref):
    h = jnp.dot(x_ref[...], w_ref[...], preferred_element_type=jnp.float32)
    s = jnp.dot(h, as_ref[...][:, None], preferred_element_type=jnp.float32)
    d = jnp.dot(h, ad_ref[...][:, None], preferred_element_type=jnp.float32)
    e = s + d
    sw = jnp.exp(jnp.where(e > 0, e, 0.2 * e))
    h_ref[...] = h
    s_ref[...] = s
    d_ref[...] = d
    sw_ref[...] = sw


def _dense_stage(xp, Wp, a_s, a_d):
    return pl.pallas_call(
        _dense_body,
        out_shape=[
            jax.ShapeDtypeStruct((N, HID), jnp.float32),
            jax.ShapeDtypeStruct((N, 1), jnp.float32),
            jax.ShapeDtypeStruct((N, 1), jnp.float32),
            jax.ShapeDtypeStruct((N, 1), jnp.float32),
        ],
    )(xp, Wp, a_s, a_d)


def _combine_body(np_ref, dp_ref, h_ref, sw_ref, b_ref, w_ref, as_ref,
                  ad_ref, h2_ref, s_ref, d_ref, sw2_ref):
    acc = np_ref[0, :N] + np_ref[1, :N]
    dcc = dp_ref[0, :N] + dp_ref[1, :N]
    sw = sw_ref[...]
    num = acc + sw * h_ref[...]
    den = dcc + sw
    z = jax.nn.relu(num / (den + 1e-16) + b_ref[...][None, :])
    h2 = jnp.dot(z, w_ref[...], preferred_element_type=jnp.float32)
    s = jnp.dot(h2, as_ref[...][:, None], preferred_element_type=jnp.float32)
    d = jnp.dot(h2, ad_ref[...][:, None], preferred_element_type=jnp.float32)
    e = s + d
    sw2 = jnp.exp(jnp.where(e > 0, e, 0.2 * e))
    h2_ref[...] = h2
    s_ref[...] = s
    d_ref[...] = d
    sw2_ref[...] = sw2


def _combine_stage(num_parts, den_parts, h, sw, b, Wn, a_s, a_d):
    return pl.pallas_call(
        _combine_body,
        out_shape=[
            jax.ShapeDtypeStruct((N, HID), jnp.float32),
            jax.ShapeDtypeStruct((N, 1), jnp.float32),
            jax.ShapeDtypeStruct((N, 1), jnp.float32),
            jax.ShapeDtypeStruct((N, 1), jnp.float32),
        ],
    )(num_parts, den_parts, h, sw, b, Wn, a_s, a_d)


GPB = 4


def _decoder_body(np_ref, dp_ref, h_ref, sw_ref, b_ref, out_ref):
    acc_all = np_ref[0] + np_ref[1]
    dcc_all = dp_ref[0] + dp_ref[1]
    for i in range(GPB):
        acc = acc_all[i * GRAPH_SIZE:(i + 1) * GRAPH_SIZE]
        dcc = dcc_all[i * GRAPH_SIZE:(i + 1) * GRAPH_SIZE]
        sw = sw_ref[i]
        num = acc + sw * h_ref[i]
        den = dcc + sw
        z = jax.nn.relu(num / (den + 1e-16) + b_ref[...][None, :])
        logits = lax.dot_general(z, z, (((1,), (1,)), ((), ())),
                                 preferred_element_type=jnp.float32)
        m = jnp.max(logits, axis=-1, keepdims=True)
        ex = jnp.exp(logits - m)
        out_ref[i] = ex / jnp.sum(ex, axis=-1, keepdims=True)


def _decoder_stage(np2, dp2, h2, sw2, b):
    grid = NB // GPB
    return pl.pallas_call(
        _decoder_body,
        grid=(grid,),
        in_specs=[
            pl.BlockSpec((2, GPB * GRAPH_SIZE, HID), lambda g: (0, g, 0)),
            pl.BlockSpec((2, GPB * GRAPH_SIZE, 1), lambda g: (0, g, 0)),
            pl.BlockSpec((GPB, GRAPH_SIZE, HID), lambda g: (g, 0, 0)),
            pl.BlockSpec((GPB, GRAPH_SIZE, 1), lambda g: (g, 0, 0)),
            pl.BlockSpec((HID,), lambda g: (0,)),
        ],
        out_specs=pl.BlockSpec((GPB, GRAPH_SIZE, GRAPH_SIZE),
                               lambda g: (g, 0, 0)),
        out_shape=jax.ShapeDtypeStruct((NB, GRAPH_SIZE, GRAPH_SIZE),
                                       jnp.float32),
    )(np2, dp2, h2, sw2, b)


def _sc_edge_body(src_hbm, dst_hbm, s_hbm, d_hbm, h_hbm, zeros_hbm,
                  num_out, den_out,
                  s_v, d_v, src_all, dst_all, hg, den_src, den_idx, colbuf,
                  rowbuf, num_sh, den_sh, sem):
    c = lax.axis_index("c")
    t = lax.axis_index("s")
    wid = c * 16 + t

    pltpu.sync_copy(s_hbm, s_v)
    pltpu.sync_copy(d_hbm, d_v)
    pltpu.sync_copy(src_hbm.at[wid], src_all)
    pltpu.sync_copy(dst_hbm.at[wid], dst_all)
    pltpu.sync_copy(zeros_hbm.at[pl.ds(0, CHUNK)], den_src)

    iota16 = lax.iota(jnp.int32, 16)
    zeros16 = jnp.zeros((16,), jnp.float32)

    def pass_body(p, pcarry):
        pltpu.sync_copy(zeros_hbm.at[pl.ds(t * ROWS_PT, ROWS_PT)],
                        num_sh.at[pl.ds(t * ROWS_PT, ROWS_PT)])
        pltpu.sync_copy(zeros_hbm.at[pl.ds(t * 8, 8)],
                        den_sh.at[pl.ds(t * 8, 8)])
        plsc.subcore_barrier()

        lo = p * HALF

        def chunk_body(i, carry):
            pltpu.async_copy(h_hbm.at[src_all.at[i]], hg, sem).wait()
            for k in range(CHUNK // 16):
                si = src_all[i, pl.ds(k * 16, 16)]
                di = dst_all[i, pl.ds(k * 16, 16)]
                sv = plsc.load_gather(s_v, [si])
                dv = plsc.load_gather(d_v, [di])
                e = sv + dv
                w = jnp.exp(jnp.where(e > 0, e, 0.2 * e))
                dl = di - lo
                owned = (dl >= 0) & (dl < HALF)
                col = jnp.bitwise_and(di, 127)
                row = jnp.where(owned, dl, HALF + col)
                drow = jnp.where(owned, lax.shift_right_logical(dl, 7),
                                 64 + jnp.bitwise_and(di, 63))
                plsc.store_scatter(den_src, [k * 16 + iota16, col], w)
                colbuf[pl.ds(k * 16, 16)] = col
                den_idx[pl.ds(k * 16, 16)] = drow
                rowbuf[pl.ds(k * 16, 16)] = row
                for l in range(16):
                    ws = w[l]
                    j = k * 16 + l
                    for r in range(HID // 16):
                        hg[j, pl.ds(r * 16, 16)] = (
                            hg[j, pl.ds(r * 16, 16)] * ws)

            pltpu.sync_copy(hg, num_sh.at[rowbuf], add=True)
            pltpu.sync_copy(den_src, den_sh.at[den_idx], add=True)

            for k in range(CHUNK // 16):
                colz = colbuf[pl.ds(k * 16, 16)]
                plsc.store_scatter(den_src, [k * 16 + iota16, colz], zeros16)
            return carry

        lax.fori_loop(0, NCHUNK, chunk_body, 0)

        plsc.subcore_barrier()

        pltpu.sync_copy(num_sh.at[pl.ds(t * ROWS_PT, ROWS_PT)],
                        num_out.at[c, p, pl.ds(t * ROWS_PT, ROWS_PT)])
        pltpu.sync_copy(den_sh.at[pl.ds(t * 8, 8)],
                        den_out.at[c, p, pl.ds(t * 8, 8)])
        return pcarry

    lax.fori_loop(0, 2, pass_body, 0)


@functools.partial(
    pl.kernel,
    out_type=(
        jax.ShapeDtypeStruct((2, 2, HALF, HID), jnp.float32),
        jax.ShapeDtypeStruct((2, 2, DROWS, 128), jnp.float32),
    ),
    mesh=plsc.VectorSubcoreMesh(core_axis_name="c", subcore_axis_name="s"),
    compiler_params=pltpu.CompilerParams(needs_layout_passes=False),
    scratch_types=[
        pltpu.VMEM((N,), jnp.float32),
        pltpu.VMEM((N,), jnp.float32),
        pltpu.VMEM((NCHUNK, CHUNK), jnp.int32),
        pltpu.VMEM((NCHUNK, CHUNK), jnp.int32),
        pltpu.VMEM((CHUNK, HID), jnp.float32),
        pltpu.VMEM((CHUNK, 128), jnp.float32),
        pltpu.VMEM((CHUNK,), jnp.int32),
        pltpu.VMEM((CHUNK,), jnp.int32),
        pltpu.VMEM((CHUNK,), jnp.int32),
        pltpu.VMEM_SHARED((ACC, HID), jnp.float32),
        pltpu.VMEM_SHARED((DROWS, 128), jnp.float32),
        pltpu.SemaphoreType.DMA,
    ],
)
def _sc_edge(src_hbm, dst_hbm, s_hbm, d_hbm, h_hbm, zeros_hbm,
             num_out, den_out,
             s_v, d_v, src_all, dst_all, hg, den_src, den_idx, colbuf,
             rowbuf, num_sh, den_sh, sem):
    _sc_edge_body(src_hbm, dst_hbm, s_hbm, d_hbm, h_hbm, zeros_hbm,
                  num_out, den_out,
                  s_v, d_v, src_all, dst_all, hg, den_src, den_idx, colbuf,
                  rowbuf, num_sh, den_sh, sem)


def kernel(x, edge_index, W1, a_src1, a_dst1, b1, W2, a_src2, a_dst2, b2):
    src = edge_index[0].astype(jnp.int32).reshape(NTILES, NCHUNK, CHUNK)
    dst = edge_index[1].astype(jnp.int32).reshape(NTILES, NCHUNK, CHUNK)
    xp = jnp.pad(x, ((0, 0), (0, HID - x.shape[1])))
    W1p = jnp.pad(W1, ((0, HID - W1.shape[0]), (0, 0)))
    zeros = jnp.zeros((HALF, HID), jnp.float32)

    h1, s1, d1, sw1 = _dense_stage(xp, W1p, a_src1, a_dst1)
    np1, dp1 = _sc_edge(src, dst, s1.reshape(N), d1.reshape(N), h1, zeros)
    h2, s2, d2, sw2 = _combine_stage(
        np1.reshape(2, NP, HID), dp1[:, :, :40, :].reshape(2, NP, 1),
        h1, sw1, b1, W2, a_src2, a_dst2)
    np2, dp2 = _sc_edge(src, dst, s2.reshape(N), d2.reshape(N), h2, zeros)
    pi = _decoder_stage(
        np2.reshape(2, NP, HID),
        dp2[:, :, :40, :].reshape(2, NP, 1),
        h2.reshape(NB, GRAPH_SIZE, HID),
        sw2.reshape(NB, GRAPH_SIZE, 1),
        b2,
    )
    return pi


# speedup vs baseline: 16.3932x; 1.1630x over previous
import functools

import jax
import jax.numpy as jnp
from jax import lax
from jax.experimental import pallas as pl
from jax.experimental.pallas import tpu as pltpu
from jax.experimental.pallas import tpu_sc as plsc

N = 10000
E = 320000
HID = 128
GRAPH_SIZE = 100
NB = N // GRAPH_SIZE

NTILES = 32
EPT = E // NTILES
CHUNK = 80
NCHUNK = EPT // CHUNK
NP = 10240
HALF = NP // 2
ACC = HALF + 128
ROWS_PT = HALF // 16
DSIZE = 8192


def _dense_body(x_ref, w_ref, as_ref, ad_ref, h_ref, s_ref, d_ref, sw_ref):
    h = jnp.dot(x_ref[...], w_ref[...], preferred_element_type=jnp.float32)
    s = jnp.dot(h, as_ref[...][:, None], preferred_element_type=jnp.float32)
    d = jnp.dot(h, ad_ref[...][:, None], preferred_element_type=jnp.float32)
    e = s + d
    sw = jnp.exp(jnp.where(e > 0, e, 0.2 * e))
    h_ref[...] = h
    s_ref[...] = s
    d_ref[...] = d
    sw_ref[...] = sw


def _dense_stage(xp, Wp, a_s, a_d):
    return pl.pallas_call(
        _dense_body,
        out_shape=[
            jax.ShapeDtypeStruct((N, HID), jnp.float32),
            jax.ShapeDtypeStruct((N, 1), jnp.float32),
            jax.ShapeDtypeStruct((N, 1), jnp.float32),
            jax.ShapeDtypeStruct((N, 1), jnp.float32),
        ],
    )(xp, Wp, a_s, a_d)


def _combine_body(np_ref, dp_ref, h_ref, sw_ref, b_ref, w_ref, as_ref,
                  ad_ref, h2_ref, s_ref, d_ref, sw2_ref):
    acc = np_ref[0, :N] + np_ref[1, :N]
    dcc = dp_ref[0, :N] + dp_ref[1, :N]
    sw = sw_ref[...]
    num = acc + sw * h_ref[...]
    den = dcc + sw
    z = jax.nn.relu(num / (den + 1e-16) + b_ref[...][None, :])
    h2 = jnp.dot(z, w_ref[...], preferred_element_type=jnp.float32)
    s = jnp.dot(h2, as_ref[...][:, None], preferred_element_type=jnp.float32)
    d = jnp.dot(h2, ad_ref[...][:, None], preferred_element_type=jnp.float32)
    e = s + d
    sw2 = jnp.exp(jnp.where(e > 0, e, 0.2 * e))
    h2_ref[...] = h2
    s_ref[...] = s
    d_ref[...] = d
    sw2_ref[...] = sw2


def _combine_stage(num_parts, den_parts, h, sw, b, Wn, a_s, a_d):
    return pl.pallas_call(
        _combine_body,
        out_shape=[
            jax.ShapeDtypeStruct((N, HID), jnp.float32),
            jax.ShapeDtypeStruct((N, 1), jnp.float32),
            jax.ShapeDtypeStruct((N, 1), jnp.float32),
            jax.ShapeDtypeStruct((N, 1), jnp.float32),
        ],
    )(num_parts, den_parts, h, sw, b, Wn, a_s, a_d)


GPB = 4


def _decoder_body(np_ref, dp_ref, h_ref, sw_ref, b_ref, out_ref):
    acc_all = np_ref[0] + np_ref[1]
    dcc_all = dp_ref[0] + dp_ref[1]
    for i in range(GPB):
        acc = acc_all[i * GRAPH_SIZE:(i + 1) * GRAPH_SIZE]
        dcc = dcc_all[i * GRAPH_SIZE:(i + 1) * GRAPH_SIZE]
        sw = sw_ref[i]
        num = acc + sw * h_ref[i]
        den = dcc + sw
        z = jax.nn.relu(num / (den + 1e-16) + b_ref[...][None, :])
        logits = lax.dot_general(z, z, (((1,), (1,)), ((), ())),
                                 preferred_element_type=jnp.float32)
        m = jnp.max(logits, axis=-1, keepdims=True)
        ex = jnp.exp(logits - m)
        out_ref[i] = ex / jnp.sum(ex, axis=-1, keepdims=True)


def _decoder_stage(np2, dp2, h2, sw2, b):
    grid = NB // GPB
    return pl.pallas_call(
        _decoder_body,
        grid=(grid,),
        in_specs=[
            pl.BlockSpec((2, GPB * GRAPH_SIZE, HID), lambda g: (0, g, 0)),
            pl.BlockSpec((2, GPB * GRAPH_SIZE, 1), lambda g: (0, g, 0)),
            pl.BlockSpec((GPB, GRAPH_SIZE, HID), lambda g: (g, 0, 0)),
            pl.BlockSpec((GPB, GRAPH_SIZE, 1), lambda g: (g, 0, 0)),
            pl.BlockSpec((HID,), lambda g: (0,)),
        ],
        out_specs=pl.BlockSpec((GPB, GRAPH_SIZE, GRAPH_SIZE),
                               lambda g: (g, 0, 0)),
        out_shape=jax.ShapeDtypeStruct((NB, GRAPH_SIZE, GRAPH_SIZE),
                                       jnp.float32),
    )(np2, dp2, h2, sw2, b)


def _sc_edge_body(src_hbm, dst_hbm, s_hbm, d_hbm, h_hbm, zeros_hbm,
                  zerod_hbm, num_out, den_out,
                  s_v, d_v, src_all, dst_all, hg, wbuf, den_idx,
                  rowbuf, num_sh, den_sh, sem):
    c = lax.axis_index("c")
    t = lax.axis_index("s")
    wid = c * 16 + t

    pltpu.sync_copy(s_hbm, s_v)
    pltpu.sync_copy(d_hbm, d_v)
    pltpu.sync_copy(src_hbm.at[wid], src_all)
    pltpu.sync_copy(dst_hbm.at[wid], dst_all)

    iota16 = lax.iota(jnp.int32, 16)
    zeros16 = jnp.zeros((16,), jnp.float32)

    def pass_body(p, pcarry):
        pltpu.sync_copy(zeros_hbm.at[pl.ds(t * ROWS_PT, ROWS_PT)],
                        num_sh.at[pl.ds(t * ROWS_PT, ROWS_PT)])
        pltpu.sync_copy(zerod_hbm.at[pl.ds(t * (DSIZE // 16), DSIZE // 16)],
                        den_sh.at[pl.ds(t * (DSIZE // 16), DSIZE // 16)])
        plsc.subcore_barrier()

        lo = p * HALF

        def chunk_body(i, carry):
            pltpu.async_copy(h_hbm.at[src_all.at[i]], hg, sem).wait()
            for k in range(CHUNK // 16):
                si = src_all[i, pl.ds(k * 16, 16)]
                di = dst_all[i, pl.ds(k * 16, 16)]
                sv = plsc.load_gather(s_v, [si])
                dv = plsc.load_gather(d_v, [di])
                e = sv + dv
                w = jnp.exp(jnp.where(e > 0, e, 0.2 * e))
                dl = di - lo
                owned = (dl >= 0) & (dl < HALF)
                col = jnp.bitwise_and(di, 127)
                row = jnp.where(owned, dl, HALF + col)
                drow = jnp.where(owned, dl, HALF + col)
                wbuf[pl.ds(k * 16, 16)] = w
                den_idx[pl.ds(k * 16, 16)] = drow
                rowbuf[pl.ds(k * 16, 16)] = row
                for l in range(16):
                    ws = w[l]
                    j = k * 16 + l
                    for r in range(HID // 16):
                        hg[j, pl.ds(r * 16, 16)] = (
                            hg[j, pl.ds(r * 16, 16)] * ws)

            pltpu.sync_copy(hg, num_sh.at[rowbuf], add=True)
            pltpu.sync_copy(wbuf, den_sh.at[den_idx], add=True)
            return carry

        lax.fori_loop(0, NCHUNK, chunk_body, 0)

        plsc.subcore_barrier()

        pltpu.sync_copy(num_sh.at[pl.ds(t * ROWS_PT, ROWS_PT)],
                        num_out.at[c, p, pl.ds(t * ROWS_PT, ROWS_PT)])
        @pl.when(t < 4)
        def _drain_den():
            pltpu.sync_copy(
                den_sh.at[pl.ds(t * 1280, 1280)],
                den_out.at[pl.ds((c * 2 + p) * HALF + t * 1280, 1280)])
        return pcarry

    lax.fori_loop(0, 2, pass_body, 0)


@functools.partial(
    pl.kernel,
    out_type=(
        jax.ShapeDtypeStruct((2, 2, HALF, HID), jnp.float32),
        jax.ShapeDtypeStruct((4 * HALF,), jnp.float32),
    ),
    mesh=plsc.VectorSubcoreMesh(core_axis_name="c", subcore_axis_name="s"),
    compiler_params=pltpu.CompilerParams(needs_layout_passes=False),
    scratch_types=[
        pltpu.VMEM((N,), jnp.float32),
        pltpu.VMEM((N,), jnp.float32),
        pltpu.VMEM((NCHUNK, CHUNK), jnp.int32),
        pltpu.VMEM((NCHUNK, CHUNK), jnp.int32),
        pltpu.VMEM((CHUNK, HID), jnp.float32),
        pltpu.VMEM((CHUNK,), jnp.float32),
        pltpu.VMEM((CHUNK,), jnp.int32),
        pltpu.VMEM((CHUNK,), jnp.int32),
        pltpu.VMEM_SHARED((ACC, HID), jnp.float32),
        pltpu.VMEM_SHARED((DSIZE,), jnp.float32),
        pltpu.SemaphoreType.DMA,
    ],
)
def _sc_edge(src_hbm, dst_hbm, s_hbm, d_hbm, h_hbm, zeros_hbm, zerod_hbm,
             num_out, den_out,
             s_v, d_v, src_all, dst_all, hg, wbuf, den_idx,
             rowbuf, num_sh, den_sh, sem):
    _sc_edge_body(src_hbm, dst_hbm, s_hbm, d_hbm, h_hbm, zeros_hbm,
                  zerod_hbm, num_out, den_out,
                  s_v, d_v, src_all, dst_all, hg, wbuf, den_idx,
                  rowbuf, num_sh, den_sh, sem)


def kernel(x, edge_index, W1, a_src1, a_dst1, b1, W2, a_src2, a_dst2, b2):
    src = edge_index[0].astype(jnp.int32).reshape(NTILES, NCHUNK, CHUNK)
    dst = edge_index[1].astype(jnp.int32).reshape(NTILES, NCHUNK, CHUNK)
    xp = jnp.pad(x, ((0, 0), (0, HID - x.shape[1])))
    W1p = jnp.pad(W1, ((0, HID - W1.shape[0]), (0, 0)))
    zeros = jnp.zeros((HALF, HID), jnp.float32)
    zerod = jnp.zeros((DSIZE,), jnp.float32)

    h1, s1, d1, sw1 = _dense_stage(xp, W1p, a_src1, a_dst1)
    np1, dp1 = _sc_edge(src, dst, s1.reshape(N), d1.reshape(N), h1, zeros, zerod)
    h2, s2, d2, sw2 = _combine_stage(
        np1.reshape(2, NP, HID), dp1.reshape(2, NP, 1),
        h1, sw1, b1, W2, a_src2, a_dst2)
    np2, dp2 = _sc_edge(src, dst, s2.reshape(N), d2.reshape(N), h2, zeros, zerod)
    pi = _decoder_stage(
        np2.reshape(2, NP, HID),
        dp2.reshape(2, NP, 1),
        h2.reshape(NB, GRAPH_SIZE, HID),
        sw2.reshape(NB, GRAPH_SIZE, 1),
        b2,
    )
    return pi


# speedup vs baseline: 27.1970x; 1.6590x over previous
import functools

import jax
import jax.numpy as jnp
from jax import lax
from jax.experimental import pallas as pl
from jax.experimental.pallas import tpu as pltpu
from jax.experimental.pallas import tpu_sc as plsc

N = 10000
E = 320000
HID = 128
GRAPH_SIZE = 100
NB = N // GRAPH_SIZE

NTILES = 32
EPT = E // NTILES
CHUNK = 80
NCHUNK = EPT // CHUNK
NP = 10240
HALF = NP // 2
ACC = HALF + 128
ROWS_PT = HALF // 16
DSIZE = 8192


def _dense_body(x_ref, w_ref, as_ref, ad_ref, h_ref, s_ref, d_ref, sw_ref):
    h = jnp.dot(x_ref[...], w_ref[...], preferred_element_type=jnp.float32)
    s = jnp.dot(h, as_ref[...][:, None], preferred_element_type=jnp.float32)
    d = jnp.dot(h, ad_ref[...][:, None], preferred_element_type=jnp.float32)
    e = s + d
    sw = jnp.exp(jnp.where(e > 0, e, 0.2 * e))
    h_ref[...] = h
    s_ref[...] = s
    d_ref[...] = d
    sw_ref[...] = sw


def _dense_stage(xp, Wp, a_s, a_d):
    return pl.pallas_call(
        _dense_body,
        out_shape=[
            jax.ShapeDtypeStruct((N, HID), jnp.float32),
            jax.ShapeDtypeStruct((N, 1), jnp.float32),
            jax.ShapeDtypeStruct((N, 1), jnp.float32),
            jax.ShapeDtypeStruct((N, 1), jnp.float32),
        ],
    )(xp, Wp, a_s, a_d)


def _combine_body(np_ref, dp_ref, h_ref, sw_ref, b_ref, w_ref, as_ref,
                  ad_ref, h2_ref, s_ref, d_ref, sw2_ref):
    acc = np_ref[0, :N] + np_ref[1, :N]
    dcc = dp_ref[0, :N] + dp_ref[1, :N]
    sw = sw_ref[...]
    num = acc + sw * h_ref[...]
    den = dcc + sw
    z = jax.nn.relu(num / (den + 1e-16) + b_ref[...][None, :])
    h2 = jnp.dot(z, w_ref[...], preferred_element_type=jnp.float32)
    s = jnp.dot(h2, as_ref[...][:, None], preferred_element_type=jnp.float32)
    d = jnp.dot(h2, ad_ref[...][:, None], preferred_element_type=jnp.float32)
    e = s + d
    sw2 = jnp.exp(jnp.where(e > 0, e, 0.2 * e))
    h2_ref[...] = h2
    s_ref[...] = s
    d_ref[...] = d
    sw2_ref[...] = sw2


def _combine_stage(num_parts, den_parts, h, sw, b, Wn, a_s, a_d):
    return pl.pallas_call(
        _combine_body,
        out_shape=[
            jax.ShapeDtypeStruct((N, HID), jnp.float32),
            jax.ShapeDtypeStruct((N, 1), jnp.float32),
            jax.ShapeDtypeStruct((N, 1), jnp.float32),
            jax.ShapeDtypeStruct((N, 1), jnp.float32),
        ],
    )(num_parts, den_parts, h, sw, b, Wn, a_s, a_d)


GPB = 4


def _decoder_body(np_ref, dp_ref, h_ref, sw_ref, b_ref, out_ref):
    acc_all = np_ref[0] + np_ref[1]
    dcc_all = dp_ref[0] + dp_ref[1]
    for i in range(GPB):
        acc = acc_all[i * GRAPH_SIZE:(i + 1) * GRAPH_SIZE]
        dcc = dcc_all[i * GRAPH_SIZE:(i + 1) * GRAPH_SIZE]
        sw = sw_ref[i]
        num = acc + sw * h_ref[i]
        den = dcc + sw
        z = jax.nn.relu(num / (den + 1e-16) + b_ref[...][None, :])
        logits = lax.dot_general(z, z, (((1,), (1,)), ((), ())),
                                 preferred_element_type=jnp.float32)
        m = jnp.max(logits, axis=-1, keepdims=True)
        ex = jnp.exp(logits - m)
        out_ref[i] = ex / jnp.sum(ex, axis=-1, keepdims=True)


def _decoder_stage(np2, dp2, h2, sw2, b):
    grid = NB // GPB
    return pl.pallas_call(
        _decoder_body,
        grid=(grid,),
        in_specs=[
            pl.BlockSpec((2, GPB * GRAPH_SIZE, HID), lambda g: (0, g, 0)),
            pl.BlockSpec((2, GPB * GRAPH_SIZE, 1), lambda g: (0, g, 0)),
            pl.BlockSpec((GPB, GRAPH_SIZE, HID), lambda g: (g, 0, 0)),
            pl.BlockSpec((GPB, GRAPH_SIZE, 1), lambda g: (g, 0, 0)),
            pl.BlockSpec((HID,), lambda g: (0,)),
        ],
        out_specs=pl.BlockSpec((GPB, GRAPH_SIZE, GRAPH_SIZE),
                               lambda g: (g, 0, 0)),
        out_shape=jax.ShapeDtypeStruct((NB, GRAPH_SIZE, GRAPH_SIZE),
                                       jnp.float32),
    )(np2, dp2, h2, sw2, b)


def _sc_edge_body(src_hbm, dst_hbm, s_hbm, d_hbm, h_hbm, zeros_hbm,
                  zerod_hbm, num_out, den_out,
                  s_v, d_v, src_all, dst_all,
                  gb0, gb1, sb, wb, rib, dib,
                  num_sh, den_sh, sg0, sg1):
    c = lax.axis_index("c")
    t = lax.axis_index("s")
    wid = c * 16 + t

    pltpu.sync_copy(s_hbm, s_v)
    pltpu.sync_copy(d_hbm, d_v)
    pltpu.sync_copy(src_hbm.at[wid], src_all)
    pltpu.sync_copy(dst_hbm.at[wid], dst_all)

    def scale_rows(gb, sb):
        for k in range(CHUNK // 16):
            wk = wb[pl.ds(k * 16, 16)]
            for l in range(16):
                ws = wk[l]
                j = k * 16 + l
                for r in range(HID // 16):
                    sb[j, pl.ds(r * 16, 16)] = (
                        gb[j, pl.ds(r * 16, 16)] * ws)

    def emit_chunk(i, lo):
        par = jnp.bitwise_and(i, 1)

        @pl.when(jnp.logical_and(par == 0, i + 1 < NCHUNK))
        def _fire1():
            pltpu.async_copy(h_hbm.at[src_all.at[i + 1]], gb1, sg1)

        @pl.when(jnp.logical_and(par == 1, i + 1 < NCHUNK))
        def _fire0():
            pltpu.async_copy(h_hbm.at[src_all.at[i + 1]], gb0, sg0)

        for k in range(CHUNK // 16):
            si = src_all[i, pl.ds(k * 16, 16)]
            di = dst_all[i, pl.ds(k * 16, 16)]
            sv = plsc.load_gather(s_v, [si])
            dv = plsc.load_gather(d_v, [di])
            e = sv + dv
            w = jnp.exp(jnp.where(e > 0, e, 0.2 * e))
            dl = di - lo
            owned = (dl >= 0) & (dl < HALF)
            tr = HALF + jnp.bitwise_and(di, 127)
            rid = jnp.where(owned, dl, tr)
            wb[pl.ds(k * 16, 16)] = w
            rib[pl.ds(k * 16, 16)] = rid
            dib[pl.ds(k * 16, 16)] = rid

        @pl.when(par == 0)
        def _scale0():
            pltpu.make_async_copy(h_hbm.at[pl.ds(0, CHUNK)], gb0, sg0).wait()
            scale_rows(gb0, sb)

        @pl.when(par == 1)
        def _scale1():
            pltpu.make_async_copy(h_hbm.at[pl.ds(0, CHUNK)], gb1, sg1).wait()
            scale_rows(gb1, sb)

        pltpu.sync_copy(sb, num_sh.at[rib], add=True)
        pltpu.sync_copy(wb, den_sh.at[dib], add=True)

    def pass_body(p, pcarry):
        pltpu.sync_copy(zeros_hbm.at[pl.ds(t * ROWS_PT, ROWS_PT)],
                        num_sh.at[pl.ds(t * ROWS_PT, ROWS_PT)])
        pltpu.sync_copy(zerod_hbm.at[pl.ds(t * (DSIZE // 16), DSIZE // 16)],
                        den_sh.at[pl.ds(t * (DSIZE // 16), DSIZE // 16)])
        plsc.subcore_barrier()

        lo = p * HALF

        pltpu.async_copy(h_hbm.at[src_all.at[0]], gb0, sg0)

        def chunk_loop(i, carry):
            emit_chunk(i, lo)
            return carry

        lax.fori_loop(0, NCHUNK, chunk_loop, 0)

        plsc.subcore_barrier()

        pltpu.sync_copy(num_sh.at[pl.ds(t * ROWS_PT, ROWS_PT)],
                        num_out.at[c, p, pl.ds(t * ROWS_PT, ROWS_PT)])

        @pl.when(t < 4)
        def _drain_den():
            pltpu.sync_copy(
                den_sh.at[pl.ds(t * 1280, 1280)],
                den_out.at[pl.ds((c * 2 + p) * HALF + t * 1280, 1280)])
        return pcarry

    lax.fori_loop(0, 2, pass_body, 0)


@functools.partial(
    pl.kernel,
    out_type=(
        jax.ShapeDtypeStruct((2, 2, HALF, HID), jnp.float32),
        jax.ShapeDtypeStruct((4 * HALF,), jnp.float32),
    ),
    mesh=plsc.VectorSubcoreMesh(core_axis_name="c", subcore_axis_name="s"),
    compiler_params=pltpu.CompilerParams(needs_layout_passes=False),
    scratch_types=[
        pltpu.VMEM((N,), jnp.float32),
        pltpu.VMEM((N,), jnp.float32),
        pltpu.VMEM((NCHUNK, CHUNK), jnp.int32),
        pltpu.VMEM((NCHUNK, CHUNK), jnp.int32),
        pltpu.VMEM((CHUNK, HID), jnp.float32),
        pltpu.VMEM((CHUNK, HID), jnp.float32),
        pltpu.VMEM((CHUNK, HID), jnp.float32),
        pltpu.VMEM((CHUNK,), jnp.float32),
        pltpu.VMEM((CHUNK,), jnp.int32),
        pltpu.VMEM((CHUNK,), jnp.int32),
        pltpu.VMEM_SHARED((ACC, HID), jnp.float32),
        pltpu.VMEM_SHARED((DSIZE,), jnp.float32),
        pltpu.SemaphoreType.DMA,
        pltpu.SemaphoreType.DMA,
    ],
)
def _sc_edge(src_hbm, dst_hbm, s_hbm, d_hbm, h_hbm, zeros_hbm, zerod_hbm,
             num_out, den_out,
             s_v, d_v, src_all, dst_all,
             gb0, gb1, sb, wb, rib, dib,
             num_sh, den_sh, sg0, sg1):
    _sc_edge_body(src_hbm, dst_hbm, s_hbm, d_hbm, h_hbm, zeros_hbm,
                  zerod_hbm, num_out, den_out,
                  s_v, d_v, src_all, dst_all,
                  gb0, gb1, sb, wb, rib, dib,
                  num_sh, den_sh, sg0, sg1)


def kernel(x, edge_index, W1, a_src1, a_dst1, b1, W2, a_src2, a_dst2, b2):
    src = edge_index[0].astype(jnp.int32).reshape(NTILES, NCHUNK, CHUNK)
    dst = edge_index[1].astype(jnp.int32).reshape(NTILES, NCHUNK, CHUNK)
    xp = jnp.pad(x, ((0, 0), (0, HID - x.shape[1])))
    W1p = jnp.pad(W1, ((0, HID - W1.shape[0]), (0, 0)))
    zeros = jnp.zeros((HALF, HID), jnp.float32)
    zerod = jnp.zeros((DSIZE,), jnp.float32)

    h1, s1, d1, sw1 = _dense_stage(xp, W1p, a_src1, a_dst1)
    np1, dp1 = _sc_edge(src, dst, s1.reshape(N), d1.reshape(N), h1, zeros, zerod)
    h2, s2, d2, sw2 = _combine_stage(
        np1.reshape(2, NP, HID), dp1.reshape(2, NP, 1),
        h1, sw1, b1, W2, a_src2, a_dst2)
    np2, dp2 = _sc_edge(src, dst, s2.reshape(N), d2.reshape(N), h2, zeros, zerod)
    pi = _decoder_stage(
        np2.reshape(2, NP, HID),
        dp2.reshape(2, NP, 1),
        h2.reshape(NB, GRAPH_SIZE, HID),
        sw2.reshape(NB, GRAPH_SIZE, 1),
        b2,
    )
    return pi
